# speedup vs baseline: 2.9612x; 2.9612x over previous
import functools

import jax
import jax.numpy as jnp
from jax import lax
from jax.experimental import pallas as pl
from jax.experimental.pallas import tpu as pltpu
from jax.experimental.pallas import tpu_sc as plsc

_NUM_TOKENS = 4096 * 50
_DIM = 128
_NC, _NS = 2, 16
_NW = _NC * _NS
_BPW = _NUM_TOKENS // _NW
_CHUNK = 128
_NCHUNK = _BPW // _CHUNK

_mesh = plsc.VectorSubcoreMesh(core_axis_name="c", subcore_axis_name="s")


@functools.partial(
    pl.kernel,
    out_type=jax.ShapeDtypeStruct((_NUM_TOKENS, _DIM), jnp.float32),
    mesh=_mesh,
    scratch_types=[
        pltpu.VMEM((_NCHUNK, _CHUNK), jnp.int32),
        pltpu.VMEM((_CHUNK, _DIM), jnp.float32),
        pltpu.SemaphoreType.DMA,
    ],
)
def _embed_gather(idx_hbm, table_hbm, out_hbm, idx_v, rows_v, sem):
    wid = lax.axis_index("s") * _NC + lax.axis_index("c")
    base = wid * _BPW
    pltpu.sync_copy(idx_hbm.at[wid], idx_v)

    @pl.loop(0, _NCHUNK)
    def _chunk(c):
        pltpu.async_copy(table_hbm.at[idx_v.at[c]], rows_v, sem).wait()
        pltpu.sync_copy(rows_v, out_hbm.at[pl.ds(base + c * _CHUNK, _CHUNK)])


def kernel(token_ids, weight):
    idx = token_ids.reshape(_NW, _NCHUNK, _CHUNK)
    out = _embed_gather(idx, weight)
    return out.reshape(*token_ids.shape, _DIM)


# speedup vs baseline: 3.3034x; 1.1155x over previous
import functools

import jax
import jax.numpy as jnp
from jax import lax
from jax.experimental import pallas as pl
from jax.experimental.pallas import tpu as pltpu
from jax.experimental.pallas import tpu_sc as plsc

_NUM_TOKENS = 4096 * 50
_DIM = 128
_NC, _NS = 2, 16
_NW = _NC * _NS
_BPW = _NUM_TOKENS // _NW
_CHUNK = 128
_NCHUNK = _BPW // _CHUNK
_NB = 5

_mesh = plsc.VectorSubcoreMesh(core_axis_name="c", subcore_axis_name="s")


@functools.partial(
    pl.kernel,
    out_type=jax.ShapeDtypeStruct((_NUM_TOKENS, _DIM), jnp.float32),
    mesh=_mesh,
    scratch_types=[
        pltpu.VMEM((_NCHUNK, _CHUNK), jnp.int32),
        pltpu.VMEM((_NB, _CHUNK, _DIM), jnp.float32),
        [pltpu.SemaphoreType.DMA] * _NB,
        [pltpu.SemaphoreType.DMA] * _NB,
    ],
)
def _embed_gather(idx_hbm, table_hbm, out_hbm, idx_v, bufs, gsems, wsems):
    wid = lax.axis_index("s") * _NC + lax.axis_index("c")
    base = wid * _BPW
    pltpu.sync_copy(idx_hbm.at[wid], idx_v)

    def fire_gather(c, b):
        pltpu.async_copy(table_hbm.at[idx_v.at[c]], bufs.at[b], gsems[b])

    def wait_gather(b):
        pltpu.make_async_copy(
            table_hbm.at[pl.ds(0, _CHUNK)], bufs.at[b], gsems[b]
        ).wait()

    def fire_write(c, b):
        pltpu.async_copy(
            bufs.at[b], out_hbm.at[pl.ds(base + c * _CHUNK, _CHUNK)], wsems[b]
        )

    def wait_write(b):
        pltpu.make_async_copy(
            bufs.at[b], out_hbm.at[pl.ds(base, _CHUNK)], wsems[b]
        ).wait()

    for b in range(_NB):
        fire_gather(b, b)

    @pl.loop(0, _NCHUNK - _NB, step=_NB)
    def _round(c0):
        for b in range(_NB):
            wait_gather(b)
            fire_write(c0 + b, b)
        for b in range(_NB):
            wait_write(b)
            fire_gather(c0 + _NB + b, b)

    for b in range(_NB):
        wait_gather(b)
        fire_write(_NCHUNK - _NB + b, b)
    for b in range(_NB):
        wait_write(b)


def kernel(token_ids, weight):
    idx = token_ids.reshape(_NW, _NCHUNK, _CHUNK)
    out = _embed_gather(idx, weight)
    return out.reshape(*token_ids.shape, _DIM)
